# baseline (speedup 1.0000x reference)
import functools

import jax
import jax.numpy as jnp
from jax import lax
from jax.experimental import pallas as pl
from jax.experimental.pallas import tpu as pltpu
from jax.experimental.pallas import tpu_sc as plsc

NC = 2
NS = 16
NW = NC * NS
CH = 40
CHS = 40


def _silu(x):
    return x * jax.nn.sigmoid(x)


def _dot(a, b):
    return lax.dot_general(a, b, (((1,), (0,)), ((), ())),
                           preferred_element_type=jnp.float32)


def _b16(v):
    r = lax.bitcast_convert_type(v, jnp.int32)
    rnd = ((lax.shift_right_logical(r, 16)) & 1) + 0x7FFF
    return lax.shift_right_logical(r + rnd, 16)


def _pack16(lo, hi):
    return _b16(lo) | lax.shift_left(_b16(hi), 16)


def _pre_body(h_ref, equ_ref, a_ref, b_ref, tr_ref, tc_ref):
    h = h_ref[...]
    equ = equ_ref[...]
    z = jnp.zeros((h.shape[0], 58), jnp.int32)
    hra = _dot(h, a_ref[...])
    hrb = _dot(h, b_ref[...])
    eq = _pack16(equ[:, 0:6], equ[:, 6:12])
    tr_ref[...] = jnp.concatenate(
        [_pack16(hra[:, 0:64], hra[:, 64:128]), eq, z], axis=1)
    tc_ref[...] = jnp.concatenate(
        [_pack16(hrb[:, 0:64], hrb[:, 64:128]), eq, z], axis=1)


def _pre(h, equ12, A, B, bn=2000):
    n = h.shape[0]
    return pl.pallas_call(
        _pre_body,
        grid=(n // bn,),
        in_specs=[pl.BlockSpec((bn, 128), lambda i: (i, 0)),
                  pl.BlockSpec((bn, 12), lambda i: (i, 0)),
                  pl.BlockSpec((128, 128), lambda i: (0, 0)),
                  pl.BlockSpec((128, 128), lambda i: (0, 0))],
        out_specs=[pl.BlockSpec((bn, 128), lambda i: (i, 0)),
                   pl.BlockSpec((bn, 128), lambda i: (i, 0))],
        out_shape=[jax.ShapeDtypeStruct((n, 128), jnp.int32),
                   jax.ShapeDtypeStruct((n, 128), jnp.int32)],
    )(h, equ12, A, B)


def _sc_gather(tr, tc, row3, col3, E):
    EW = E // NW
    NCH = EW // CH
    assert NCH % 2 == 0
    mesh = plsc.VectorSubcoreMesh(core_axis_name="c", subcore_axis_name="s")

    @functools.partial(
        pl.kernel, mesh=mesh,
        out_type=jax.ShapeDtypeStruct((E, 80), jnp.int32),
        scratch_types=[pltpu.VMEM((NCH, CH), jnp.int32),
                       pltpu.VMEM((NCH, CH), jnp.int32),
                       pltpu.VMEM((2, CH, 128), jnp.int32),
                       pltpu.VMEM((2, CH, 128), jnp.int32),
                       pltpu.VMEM((CH, 80), jnp.int32),
                       pltpu.SemaphoreType.DMA,
                       pltpu.SemaphoreType.DMA],
    )
    def k(tr_hbm, tc_hbm, row_hbm, col_hbm, g_hbm,
          idxr_v, idxc_v, buf_a, buf_b, buf_g, semr0, semr1):
        wid = lax.axis_index("s") * NC + lax.axis_index("c")
        pltpu.sync_copy(row_hbm.at[wid], idxr_v)
        pltpu.sync_copy(col_hbm.at[wid], idxc_v)
        semr = (semr0, semr1)

        def issue_reads(j, s):
            pltpu.async_copy(tr_hbm.at[idxr_v.at[j]], buf_a.at[s], semr[s])
            pltpu.async_copy(tc_hbm.at[idxc_v.at[j]], buf_b.at[s], semr[s])

        def wait_reads(j, s):
            pltpu.make_async_copy(tr_hbm.at[idxr_v.at[j]], buf_a.at[s],
                                  semr[s]).wait()
            pltpu.make_async_copy(tc_hbm.at[idxc_v.at[j]], buf_b.at[s],
                                  semr[s]).wait()

        issue_reads(0, 0)
        issue_reads(1, 1)

        def body2(jj, carry):
            j0 = jj * 2
            process_dyn(j0, 0, jj)
            process_dyn(j0 + 1, 1, jj)
            return carry

        def process_dyn(j, s, jj):
            wait_reads(j, s)

            @pl.when(j + 2 < NCH)
            def _():
                issue_reads(j + 2, s)

            ba, bb = buf_a.at[s], buf_b.at[s]
            f32 = jnp.float32
            i32 = jnp.int32
            m16 = jnp.int32(-65536)

            def unpk(w):
                lo = lax.bitcast_convert_type(lax.shift_left(w, 16), f32)
                hi = lax.bitcast_convert_type(w & m16, f32)
                return lo, hi

            def repk(lo, hi):
                wl = lax.shift_right_logical(
                    lax.bitcast_convert_type(lo, i32), 16)
                return wl | (lax.bitcast_convert_type(hi, i32) & m16)

            @plsc.parallel_loop(0, CH, step=1, unroll=4)
            def _(i):
                for kk in range(4):
                    sl = pl.ds(kk * 16, 16)
                    alo, ahi = unpk(ba[i, sl])
                    blo, bhi = unpk(bb[i, sl])
                    buf_g[i, sl] = repk(alo + blo, ahi + bhi)
                sle = pl.ds(64, 16)
                alo, ahi = unpk(ba[i, sle])
                blo, bhi = unpk(bb[i, sle])
                buf_g[i, sle] = repk(alo - blo, ahi - bhi)

            base = wid * EW + j * CH
            pltpu.sync_copy(buf_g, g_hbm.at[pl.ds(base, CH)])

        lax.fori_loop(0, NCH // 2, body2, 0)

    return k(tr, tc, row3, col3)


def _unpk_lo(w):
    return lax.bitcast_convert_type(lax.shift_left(w, 16), jnp.float32)


def _unpk_hi(w):
    return lax.bitcast_convert_type(w & jnp.int32(-65536), jnp.float32)


def _edge_body(g_ref, ef_ref, waux_ref, be1_ref, we2_ref, be2_ref,
               wc1_ref, bc1_ref, wc2r_ref, bc2_ref, msg_ref, aux_ref):
    x = g_ref[...]
    xg = x[:, 0:64]
    g = jnp.concatenate([_unpk_lo(xg), _unpk_hi(xg)], axis=1)
    xr = x[:, 64:70]
    rij = jnp.concatenate([_unpk_lo(xr), _unpk_hi(xr)], axis=1)
    norms = jnp.sqrt(rij[:, 0:4] ** 2 + rij[:, 4:8] ** 2 + rij[:, 8:12] ** 2)
    nf = jnp.concatenate([norms, ef_ref[...]], axis=1)
    y1 = g + _dot(nf, waux_ref[...]) + be1_ref[...]
    a1 = _silu(y1)
    msg = _silu(_dot(a1, we2_ref[...]) + be2_ref[...])
    msg_ref[...] = msg
    a2 = _silu(_dot(msg, wc1_ref[...]) + bc1_ref[...])
    coord = jnp.sum(a2 * wc2r_ref[...], axis=1, keepdims=True) + bc2_ref[...]
    f12 = rij * coord
    one = jnp.ones((f12.shape[0], 1), jnp.float32)
    z3 = jnp.zeros((f12.shape[0], 3), jnp.float32)
    aux_ref[...] = jnp.concatenate([f12, one, z3], axis=1)


def _edge(g, ef, waux, be1, we2, be2, wc1, bc1, wc2r, bc2, be=1280):
    E = g.shape[0]
    full = lambda r, c: pl.BlockSpec((r, c), lambda i: (0, 0))
    return pl.pallas_call(
        _edge_body,
        grid=(E // be,),
        in_specs=[pl.BlockSpec((be, 80), lambda i: (i, 0)),
                  pl.BlockSpec((be, 16), lambda i: (i, 0)),
                  full(20, 128), full(1, 128), full(128, 128), full(1, 128),
                  full(128, 128), full(1, 128), full(1, 128), full(1, 1)],
        out_specs=[pl.BlockSpec((be, 128), lambda i: (i, 0)),
                   pl.BlockSpec((be, 16), lambda i: (i, 0))],
        out_shape=[jax.ShapeDtypeStruct((E, 128), jnp.float32),
                   jax.ShapeDtypeStruct((E, 16), jnp.float32)],
    )(g, ef, waux, be1, we2, be2, wc1, bc1, wc2r, bc2)


def _sc_scatter(msg, aux, row3, init3, E, N):
    EW = E // NS
    NCHS = EW // CHS
    assert NCHS % 2 == 0
    RPT = (N // (8 * NS)) * 8
    REM = N - RPT * NS
    mesh = plsc.VectorSubcoreMesh(core_axis_name="c", subcore_axis_name="s")

    @functools.partial(
        pl.kernel, mesh=mesh,
        out_type=jax.ShapeDtypeStruct((NC, N, 128), jnp.float32),
        scratch_types=[pltpu.VMEM((2, CHS), jnp.int32),
                       pltpu.VMEM((2, CHS, 128), jnp.float32),
                       pltpu.VMEM((2, CHS, 16), jnp.float32),
                       pltpu.VMEM((CHS, 128), jnp.float32),
                       pltpu.VMEM_SHARED((N, 128), jnp.float32),
                       pltpu.SemaphoreType.DMA,
                       pltpu.SemaphoreType.DMA],
    )
    def k(msg_hbm, aux_hbm, row_hbm, z_hbm, acc_hbm, idx_v, vbuf, vbuf16,
          vbuff, acc, sem0, sem1):
        cid = lax.axis_index("c")
        sid = lax.axis_index("s")
        r0 = sid * RPT
        sem = (sem0, sem1)

        def over_rows(fn):
            fn(r0, RPT)
            if REM:
                @pl.when(sid == NS - 1)
                def _():
                    fn(RPT * NS, REM)

        over_rows(lambda o, n: pltpu.sync_copy(z_hbm.at[cid, pl.ds(o, n)],
                                               acc.at[pl.ds(o, n)]))
        plsc.subcore_barrier()

        def scatter_msg():
            def issue(j, s):
                base = sid * EW + j * CHS
                pltpu.async_copy(msg_hbm.at[pl.ds(base, CHS)], vbuf.at[s],
                                 sem[s])
                pltpu.async_copy(row_hbm.at[sid, j], idx_v.at[s], sem[s])

            def wait_read(j, s):
                base = sid * EW + j * CHS
                pltpu.make_async_copy(msg_hbm.at[pl.ds(base, CHS)],
                                      vbuf.at[s], sem[s]).wait()
                pltpu.make_async_copy(row_hbm.at[sid, j], idx_v.at[s],
                                      sem[s]).wait()

            issue(0, 0)
            issue(1, 1)

            def body2(jj, carry):
                j0 = jj * 2
                for s in (0, 1):
                    j = j0 + s
                    wait_read(j, s)
                    pltpu.sync_copy(vbuf.at[s], acc.at[idx_v.at[s]], add=True)

                    @pl.when(j + 2 < NCHS)
                    def _():
                        issue(j + 2, s)
                return carry

            lax.fori_loop(0, NCHS // 2, body2, 0)

        def scatter_aux():
            @plsc.parallel_loop(0, CHS, step=1, unroll=4)
            def _(i):
                z16 = jnp.zeros((16,), jnp.float32)
                for kk in range(8):
                    vbuff[i, pl.ds(kk * 16, 16)] = z16

            def issue(j, s):
                base = sid * EW + j * CHS
                pltpu.async_copy(aux_hbm.at[pl.ds(base, CHS)], vbuf16.at[s],
                                 sem[s])
                pltpu.async_copy(row_hbm.at[sid, j], idx_v.at[s], sem[s])

            def wait_read(j, s):
                base = sid * EW + j * CHS
                pltpu.make_async_copy(aux_hbm.at[pl.ds(base, CHS)],
                                      vbuf16.at[s], sem[s]).wait()
                pltpu.make_async_copy(row_hbm.at[sid, j], idx_v.at[s],
                                      sem[s]).wait()

            issue(0, 0)
            issue(1, 1)

            def body2(jj, carry):
                j0 = jj * 2
                for s in (0, 1):
                    j = j0 + s
                    wait_read(j, s)

                    @plsc.parallel_loop(0, CHS, step=1, unroll=4)
                    def _(i):
                        vbuff[i, pl.ds(0, 16)] = vbuf16[s, i, pl.ds(0, 16)]

                    pltpu.sync_copy(vbuff, acc.at[idx_v.at[s]], add=True)

                    @pl.when(j + 2 < NCHS)
                    def _():
                        issue(j + 2, s)
                return carry

            lax.fori_loop(0, NCHS // 2, body2, 0)

        @pl.when(cid == 0)
        def _():
            scatter_msg()

        @pl.when(cid == 1)
        def _():
            scatter_aux()

        plsc.subcore_barrier()
        over_rows(lambda o, n: pltpu.sync_copy(acc.at[pl.ds(o, n)],
                                               acc_hbm.at[cid, pl.ds(o, n)]))

    return k(msg, aux, row3, init3)


def _node_body(h_ref, equ_ref, acc_ref, wq1_ref, bq1_ref, wq2r_ref,
               bq2_ref, wn1a_ref, wn1b_ref, bn1_ref, wn2_ref, bn2_ref,
               equo_ref, hout_ref):
    h = h_ref[...]
    tm = acc_ref[0]
    s2 = acc_ref[1]
    cnt = jnp.maximum(s2[:, 12:13], 1.0)
    totf = jnp.clip(s2[:, 0:12] / cnt, -100.0, 100.0)
    aq = _silu(_dot(h, wq1_ref[...]) + bq1_ref[...])
    gate = jnp.sum(aq * wq2r_ref[...], axis=1, keepdims=True) + bq2_ref[...]
    equo_ref[...] = gate * equ_ref[...] + totf
    nb = _silu(_dot(h, wn1a_ref[...]) + _dot(tm, wn1b_ref[...]) + bn1_ref[...])
    hout_ref[...] = _dot(nb, wn2_ref[...]) + bn2_ref[...]


def _node(h, equ12, acc, wq1, bq1, wq2r, bq2, wn1a, wn1b, bn1, wn2,
          bn2, bn=2000):
    N = h.shape[0]
    full = lambda r, c: pl.BlockSpec((r, c), lambda i: (0, 0))
    return pl.pallas_call(
        _node_body,
        grid=(N // bn,),
        in_specs=[pl.BlockSpec((bn, 128), lambda i: (i, 0)),
                  pl.BlockSpec((bn, 12), lambda i: (i, 0)),
                  pl.BlockSpec((2, bn, 128), lambda i: (0, i, 0)),
                  full(128, 128), full(1, 128), full(1, 128), full(1, 1),
                  full(128, 128), full(128, 128), full(1, 128),
                  full(128, 128), full(1, 128)],
        out_specs=[pl.BlockSpec((bn, 12), lambda i: (i, 0)),
                   pl.BlockSpec((bn, 128), lambda i: (i, 0))],
        out_shape=[jax.ShapeDtypeStruct((N, 12), jnp.float32),
                   jax.ShapeDtypeStruct((N, 128), jnp.float32)],
    )(h, equ12, acc, wq1, bq1, wq2r, bq2, wn1a, wn1b, bn1, wn2, bn2)


def kernel(equ, h, edge_fea, w_e1, b_e1, w_e2, b_e2, w_c1, b_c1, w_c2, b_c2,
           w_n1, b_n1, w_n2, b_n2, w_q1, b_q1, w_q2, b_q2, edge_index):
    N = h.shape[0]
    E = edge_fea.shape[0]
    M = equ.shape[2]

    equ12 = equ.reshape(N, 3 * M)
    row = edge_index[0]
    col = edge_index[1]

    waux = jnp.concatenate([w_e1[0:4], w_e1[260:276]], axis=0)
    r1 = lambda b: b.reshape(1, -1)

    tr, tcb = _pre(h, equ12, w_e1[4:132], w_e1[132:260])

    U = E // 2560
    units = [(U * 18) // 100, (U * 23) // 100, (U * 23) // 100,
             (U * 22) // 100, 0]
    units[4] = U - sum(units[:4])
    sizes = [u * 2560 for u in units]
    splits = []
    lo = 0
    for sz in sizes:
        splits.append((lo, lo + sz))
        lo += sz
    gs = []
    for (lo, hi) in splits:
        Eh = hi - lo
        r3 = lax.slice_in_dim(row, lo, hi).reshape(NW, Eh // NW // CH, CH)
        c3 = lax.slice_in_dim(col, lo, hi).reshape(NW, Eh // NW // CH, CH)
        gs.append(_sc_gather(tr, tcb, r3, c3, Eh))
    ms = []
    for (lo, hi), g in zip(splits, gs):
        ef = lax.slice_in_dim(edge_fea, lo, hi)
        ms.append(_edge(g, ef, waux, r1(b_e1), w_e2, r1(b_e2), w_c1,
                        r1(b_c1), w_c2.reshape(1, 128), b_c2.reshape(1, 1)))
    acc = jnp.zeros((NC, N, 128), jnp.float32)
    for (lo, hi), (msg, aux) in zip(splits, ms):
        Eh = hi - lo
        r3s = lax.slice_in_dim(row, lo, hi).reshape(NS, Eh // NS // CHS, CHS)
        acc = _sc_scatter(msg, aux, r3s, acc, Eh, N)
    equo, h_out = _node(h, equ12, acc, w_q1, r1(b_q1),
                        w_q2.reshape(1, 128), b_q2.reshape(1, 1),
                        w_n1[:128], w_n1[128:], r1(b_n1), w_n2, r1(b_n2))
    return equo.reshape(N, 3, M), h_out


# speedup vs baseline: 1.0193x; 1.0193x over previous
import functools

import jax
import jax.numpy as jnp
from jax import lax
from jax.experimental import pallas as pl
from jax.experimental.pallas import tpu as pltpu
from jax.experimental.pallas import tpu_sc as plsc

NC = 2
NS = 16
NW = NC * NS
CH = 40
CHS = 40


def _silu(x):
    return x * jax.nn.sigmoid(x)


def _dot(a, b):
    return lax.dot_general(a, b, (((1,), (0,)), ((), ())),
                           preferred_element_type=jnp.float32)


def _b16(v):
    r = lax.bitcast_convert_type(v, jnp.int32)
    rnd = ((lax.shift_right_logical(r, 16)) & 1) + 0x7FFF
    return lax.shift_right_logical(r + rnd, 16)


def _pack16(lo, hi):
    return _b16(lo) | lax.shift_left(_b16(hi), 16)


def _pre_body(h_ref, equ_ref, a_ref, b_ref, tr_ref, tc_ref):
    h = h_ref[...]
    equ = equ_ref[...]
    z = jnp.zeros((h.shape[0], 58), jnp.int32)
    hra = _dot(h, a_ref[...])
    hrb = _dot(h, b_ref[...])
    eq = _pack16(equ[:, 0:6], equ[:, 6:12])
    tr_ref[...] = jnp.concatenate(
        [_pack16(hra[:, 0:64], hra[:, 64:128]), eq, z], axis=1)
    tc_ref[...] = jnp.concatenate(
        [_pack16(hrb[:, 0:64], hrb[:, 64:128]), eq, z], axis=1)


def _pre(h, equ12, A, B, bn=2000):
    n = h.shape[0]
    return pl.pallas_call(
        _pre_body,
        grid=(n // bn,),
        in_specs=[pl.BlockSpec((bn, 128), lambda i: (i, 0)),
                  pl.BlockSpec((bn, 12), lambda i: (i, 0)),
                  pl.BlockSpec((128, 128), lambda i: (0, 0)),
                  pl.BlockSpec((128, 128), lambda i: (0, 0))],
        out_specs=[pl.BlockSpec((bn, 128), lambda i: (i, 0)),
                   pl.BlockSpec((bn, 128), lambda i: (i, 0))],
        out_shape=[jax.ShapeDtypeStruct((n, 128), jnp.int32),
                   jax.ShapeDtypeStruct((n, 128), jnp.int32)],
    )(h, equ12, A, B)


def _sc_gather(tr, tc, row3, col3, E):
    EW = E // NW
    NCH = EW // CH
    assert NCH % 2 == 0
    mesh = plsc.VectorSubcoreMesh(core_axis_name="c", subcore_axis_name="s")

    @functools.partial(
        pl.kernel, mesh=mesh,
        out_type=jax.ShapeDtypeStruct((E, 80), jnp.int32),
        scratch_types=[pltpu.VMEM((NCH, CH), jnp.int32),
                       pltpu.VMEM((NCH, CH), jnp.int32),
                       pltpu.VMEM((2, CH, 128), jnp.int32),
                       pltpu.VMEM((2, CH, 128), jnp.int32),
                       pltpu.VMEM((CH, 80), jnp.int32),
                       pltpu.SemaphoreType.DMA,
                       pltpu.SemaphoreType.DMA],
    )
    def k(tr_hbm, tc_hbm, row_hbm, col_hbm, g_hbm,
          idxr_v, idxc_v, buf_a, buf_b, buf_g, semr0, semr1):
        wid = lax.axis_index("s") * NC + lax.axis_index("c")
        pltpu.sync_copy(row_hbm.at[wid], idxr_v)
        pltpu.sync_copy(col_hbm.at[wid], idxc_v)
        semr = (semr0, semr1)

        def issue_reads(j, s):
            pltpu.async_copy(tr_hbm.at[idxr_v.at[j]], buf_a.at[s], semr[s])
            pltpu.async_copy(tc_hbm.at[idxc_v.at[j]], buf_b.at[s], semr[s])

        def wait_reads(j, s):
            pltpu.make_async_copy(tr_hbm.at[idxr_v.at[j]], buf_a.at[s],
                                  semr[s]).wait()
            pltpu.make_async_copy(tc_hbm.at[idxc_v.at[j]], buf_b.at[s],
                                  semr[s]).wait()

        issue_reads(0, 0)
        issue_reads(1, 1)

        def body2(jj, carry):
            j0 = jj * 2
            process_dyn(j0, 0, jj)
            process_dyn(j0 + 1, 1, jj)
            return carry

        def process_dyn(j, s, jj):
            wait_reads(j, s)

            @pl.when(j + 2 < NCH)
            def _():
                issue_reads(j + 2, s)

            ba, bb = buf_a.at[s], buf_b.at[s]
            f32 = jnp.float32
            i32 = jnp.int32
            m16 = jnp.int32(-65536)

            def unpk(w):
                lo = lax.bitcast_convert_type(lax.shift_left(w, 16), f32)
                hi = lax.bitcast_convert_type(w & m16, f32)
                return lo, hi

            def repk(lo, hi):
                wl = lax.shift_right_logical(
                    lax.bitcast_convert_type(lo, i32), 16)
                return wl | (lax.bitcast_convert_type(hi, i32) & m16)

            @plsc.parallel_loop(0, CH, step=1, unroll=4)
            def _(i):
                for kk in range(4):
                    sl = pl.ds(kk * 16, 16)
                    alo, ahi = unpk(ba[i, sl])
                    blo, bhi = unpk(bb[i, sl])
                    buf_g[i, sl] = repk(alo + blo, ahi + bhi)
                sle = pl.ds(64, 16)
                alo, ahi = unpk(ba[i, sle])
                blo, bhi = unpk(bb[i, sle])
                buf_g[i, sle] = repk(alo - blo, ahi - bhi)

            base = wid * EW + j * CH
            pltpu.sync_copy(buf_g, g_hbm.at[pl.ds(base, CH)])

        lax.fori_loop(0, NCH // 2, body2, 0)

    return k(tr, tc, row3, col3)


def _unpk_lo(w):
    return lax.bitcast_convert_type(lax.shift_left(w, 16), jnp.float32)


def _unpk_hi(w):
    return lax.bitcast_convert_type(w & jnp.int32(-65536), jnp.float32)


def _edge_body(g_ref, ef_ref, waux_ref, be1_ref, we2_ref, be2_ref,
               wc1_ref, bc1_ref, wc2r_ref, bc2_ref, msg_ref, aux_ref):
    x = g_ref[...]
    xg = x[:, 0:64]
    g = jnp.concatenate([_unpk_lo(xg), _unpk_hi(xg)], axis=1)
    xr = x[:, 64:70]
    rij = jnp.concatenate([_unpk_lo(xr), _unpk_hi(xr)], axis=1)
    norms = jnp.sqrt(rij[:, 0:4] ** 2 + rij[:, 4:8] ** 2 + rij[:, 8:12] ** 2)
    nf = jnp.concatenate([norms, ef_ref[...]], axis=1)
    y1 = g + _dot(nf, waux_ref[...]) + be1_ref[...]
    a1 = _silu(y1)
    msg = _silu(_dot(a1, we2_ref[...]) + be2_ref[...])
    msg_ref[...] = msg
    a2 = _silu(_dot(msg, wc1_ref[...]) + bc1_ref[...])
    coord = jnp.sum(a2 * wc2r_ref[...], axis=1, keepdims=True) + bc2_ref[...]
    f12 = rij * coord
    one = jnp.ones((f12.shape[0], 1), jnp.float32)
    z3 = jnp.zeros((f12.shape[0], 3), jnp.float32)
    aux_ref[...] = jnp.concatenate([f12, one, z3], axis=1)


def _edge(g, ef, waux, be1, we2, be2, wc1, bc1, wc2r, bc2, be=1280):
    E = g.shape[0]
    full = lambda r, c: pl.BlockSpec((r, c), lambda i: (0, 0))
    return pl.pallas_call(
        _edge_body,
        grid=(E // be,),
        in_specs=[pl.BlockSpec((be, 80), lambda i: (i, 0)),
                  pl.BlockSpec((be, 16), lambda i: (i, 0)),
                  full(20, 128), full(1, 128), full(128, 128), full(1, 128),
                  full(128, 128), full(1, 128), full(1, 128), full(1, 1)],
        out_specs=[pl.BlockSpec((be, 128), lambda i: (i, 0)),
                   pl.BlockSpec((be, 16), lambda i: (i, 0))],
        out_shape=[jax.ShapeDtypeStruct((E, 128), jnp.float32),
                   jax.ShapeDtypeStruct((E, 16), jnp.float32)],
    )(g, ef, waux, be1, we2, be2, wc1, bc1, wc2r, bc2)


def _sc_scatter(msg, aux, row3, init3, E, N):
    EW = E // NS
    NCHS = EW // CHS
    assert NCHS % 2 == 0
    RPT = (N // (8 * NS)) * 8
    REM = N - RPT * NS
    mesh = plsc.VectorSubcoreMesh(core_axis_name="c", subcore_axis_name="s")

    @functools.partial(
        pl.kernel, mesh=mesh,
        out_type=jax.ShapeDtypeStruct((NC, N, 128), jnp.float32),
        scratch_types=[pltpu.VMEM((2, CHS), jnp.int32),
                       pltpu.VMEM((2, CHS, 128), jnp.float32),
                       pltpu.VMEM((2, CHS, 16), jnp.float32),
                       pltpu.VMEM((CHS, 128), jnp.float32),
                       pltpu.VMEM_SHARED((N, 128), jnp.float32),
                       pltpu.SemaphoreType.DMA,
                       pltpu.SemaphoreType.DMA],
    )
    def k(msg_hbm, aux_hbm, row_hbm, z_hbm, acc_hbm, idx_v, vbuf, vbuf16,
          vbuff, acc, sem0, sem1):
        cid = lax.axis_index("c")
        sid = lax.axis_index("s")
        r0 = sid * RPT
        sem = (sem0, sem1)

        def over_rows(fn):
            fn(r0, RPT)
            if REM:
                @pl.when(sid == NS - 1)
                def _():
                    fn(RPT * NS, REM)

        over_rows(lambda o, n: pltpu.sync_copy(z_hbm.at[cid, pl.ds(o, n)],
                                               acc.at[pl.ds(o, n)]))
        plsc.subcore_barrier()

        def scatter_msg():
            def issue(j, s):
                base = sid * EW + j * CHS
                pltpu.async_copy(msg_hbm.at[pl.ds(base, CHS)], vbuf.at[s],
                                 sem[s])
                pltpu.async_copy(row_hbm.at[sid, j], idx_v.at[s], sem[s])

            def wait_read(j, s):
                base = sid * EW + j * CHS
                pltpu.make_async_copy(msg_hbm.at[pl.ds(base, CHS)],
                                      vbuf.at[s], sem[s]).wait()
                pltpu.make_async_copy(row_hbm.at[sid, j], idx_v.at[s],
                                      sem[s]).wait()

            issue(0, 0)
            issue(1, 1)

            def body2(jj, carry):
                j0 = jj * 2
                for s in (0, 1):
                    j = j0 + s
                    wait_read(j, s)
                    pltpu.sync_copy(vbuf.at[s], acc.at[idx_v.at[s]], add=True)

                    @pl.when(j + 2 < NCHS)
                    def _():
                        issue(j + 2, s)
                return carry

            lax.fori_loop(0, NCHS // 2, body2, 0)

        def scatter_aux():
            @plsc.parallel_loop(0, CHS, step=1, unroll=4)
            def _(i):
                z16 = jnp.zeros((16,), jnp.float32)
                for kk in range(8):
                    vbuff[i, pl.ds(kk * 16, 16)] = z16

            def issue(j, s):
                base = sid * EW + j * CHS
                pltpu.async_copy(aux_hbm.at[pl.ds(base, CHS)], vbuf16.at[s],
                                 sem[s])
                pltpu.async_copy(row_hbm.at[sid, j], idx_v.at[s], sem[s])

            def wait_read(j, s):
                base = sid * EW + j * CHS
                pltpu.make_async_copy(aux_hbm.at[pl.ds(base, CHS)],
                                      vbuf16.at[s], sem[s]).wait()
                pltpu.make_async_copy(row_hbm.at[sid, j], idx_v.at[s],
                                      sem[s]).wait()

            issue(0, 0)
            issue(1, 1)

            def body2(jj, carry):
                j0 = jj * 2
                for s in (0, 1):
                    j = j0 + s
                    wait_read(j, s)

                    @plsc.parallel_loop(0, CHS, step=1, unroll=4)
                    def _(i):
                        vbuff[i, pl.ds(0, 16)] = vbuf16[s, i, pl.ds(0, 16)]

                    pltpu.sync_copy(vbuff, acc.at[idx_v.at[s]], add=True)

                    @pl.when(j + 2 < NCHS)
                    def _():
                        issue(j + 2, s)
                return carry

            lax.fori_loop(0, NCHS // 2, body2, 0)

        @pl.when(cid == 0)
        def _():
            scatter_msg()

        @pl.when(cid == 1)
        def _():
            scatter_aux()

        plsc.subcore_barrier()
        over_rows(lambda o, n: pltpu.sync_copy(acc.at[pl.ds(o, n)],
                                               acc_hbm.at[cid, pl.ds(o, n)]))

    return k(msg, aux, row3, init3)


def _node_body(h_ref, equ_ref, acc_ref, wq1_ref, bq1_ref, wq2r_ref,
               bq2_ref, wn1a_ref, wn1b_ref, bn1_ref, wn2_ref, bn2_ref,
               equo_ref, hout_ref):
    h = h_ref[...]
    tm = acc_ref[0]
    s2 = acc_ref[1]
    cnt = jnp.maximum(s2[:, 12:13], 1.0)
    totf = jnp.clip(s2[:, 0:12] / cnt, -100.0, 100.0)
    aq = _silu(_dot(h, wq1_ref[...]) + bq1_ref[...])
    gate = jnp.sum(aq * wq2r_ref[...], axis=1, keepdims=True) + bq2_ref[...]
    equo_ref[...] = gate * equ_ref[...] + totf
    nb = _silu(_dot(h, wn1a_ref[...]) + _dot(tm, wn1b_ref[...]) + bn1_ref[...])
    hout_ref[...] = _dot(nb, wn2_ref[...]) + bn2_ref[...]


def _node(h, equ12, acc, wq1, bq1, wq2r, bq2, wn1a, wn1b, bn1, wn2,
          bn2, bn=2000):
    N = h.shape[0]
    full = lambda r, c: pl.BlockSpec((r, c), lambda i: (0, 0))
    return pl.pallas_call(
        _node_body,
        grid=(N // bn,),
        in_specs=[pl.BlockSpec((bn, 128), lambda i: (i, 0)),
                  pl.BlockSpec((bn, 12), lambda i: (i, 0)),
                  pl.BlockSpec((2, bn, 128), lambda i: (0, i, 0)),
                  full(128, 128), full(1, 128), full(1, 128), full(1, 1),
                  full(128, 128), full(128, 128), full(1, 128),
                  full(128, 128), full(1, 128)],
        out_specs=[pl.BlockSpec((bn, 12), lambda i: (i, 0)),
                   pl.BlockSpec((bn, 128), lambda i: (i, 0))],
        out_shape=[jax.ShapeDtypeStruct((N, 12), jnp.float32),
                   jax.ShapeDtypeStruct((N, 128), jnp.float32)],
    )(h, equ12, acc, wq1, bq1, wq2r, bq2, wn1a, wn1b, bn1, wn2, bn2)


def kernel(equ, h, edge_fea, w_e1, b_e1, w_e2, b_e2, w_c1, b_c1, w_c2, b_c2,
           w_n1, b_n1, w_n2, b_n2, w_q1, b_q1, w_q2, b_q2, edge_index):
    N = h.shape[0]
    E = edge_fea.shape[0]
    M = equ.shape[2]

    equ12 = equ.reshape(N, 3 * M)
    row = edge_index[0]
    col = edge_index[1]

    waux = jnp.concatenate([w_e1[0:4], w_e1[260:276]], axis=0)
    r1 = lambda b: b.reshape(1, -1)

    tr, tcb = _pre(h, equ12, w_e1[4:132], w_e1[132:260])

    U = E // 2560
    units = [(U * 20) // 100, (U * 30) // 100, (U * 30) // 100, 0]
    units[3] = U - sum(units[:3])
    sizes = [u * 2560 for u in units]
    splits = []
    lo = 0
    for sz in sizes:
        splits.append((lo, lo + sz))
        lo += sz
    gs = []
    for (lo, hi) in splits:
        Eh = hi - lo
        r3 = lax.slice_in_dim(row, lo, hi).reshape(NW, Eh // NW // CH, CH)
        c3 = lax.slice_in_dim(col, lo, hi).reshape(NW, Eh // NW // CH, CH)
        gs.append(_sc_gather(tr, tcb, r3, c3, Eh))
    ms = []
    for (lo, hi), g in zip(splits, gs):
        ef = lax.slice_in_dim(edge_fea, lo, hi)
        ms.append(_edge(g, ef, waux, r1(b_e1), w_e2, r1(b_e2), w_c1,
                        r1(b_c1), w_c2.reshape(1, 128), b_c2.reshape(1, 1)))
    acc = jnp.zeros((NC, N, 128), jnp.float32)
    for (lo, hi), (msg, aux) in zip(splits, ms):
        Eh = hi - lo
        r3s = lax.slice_in_dim(row, lo, hi).reshape(NS, Eh // NS // CHS, CHS)
        acc = _sc_scatter(msg, aux, r3s, acc, Eh, N)
    equo, h_out = _node(h, equ12, acc, w_q1, r1(b_q1),
                        w_q2.reshape(1, 128), b_q2.reshape(1, 1),
                        w_n1[:128], w_n1[128:], r1(b_n1), w_n2, r1(b_n2))
    return equo.reshape(N, 3, M), h_out
